# Initial kernel scaffold; baseline (speedup 1.0000x reference)
import functools

import jax
import jax.numpy as jnp
from jax import lax
from jax.experimental import pallas as pl
from jax.experimental.pallas import tpu as pltpu
from jax.experimental.pallas import tpu_sc as plsc

NC = 2
NS = 16
NW = NC * NS
CHUNK = 128

F32 = jnp.float32


def _proj_body(h_ref, b_ref, m_ref, ws_ref, hs_ref, sp_ref):
    h = h_ref[...]
    hb = jnp.dot(h, b_ref[...], preferred_element_type=F32)
    hs_ref[...] = jnp.dot(hb, m_ref[...], preferred_element_type=F32)
    sp_ref[...] = jnp.dot(h, ws_ref[...], preferred_element_type=F32)


def _proj2_body(p_ref, sp1_ref, b_ref, m_ref, ws_ref, hs_ref, sp_ref):
    h = jnp.maximum(p_ref[0] + p_ref[1] + sp1_ref[...], 0.0)
    hb = jnp.dot(h, b_ref[...], preferred_element_type=F32)
    hs_ref[...] = jnp.dot(hb, m_ref[...], preferred_element_type=F32)
    sp_ref[...] = jnp.dot(h, ws_ref[...], preferred_element_type=F32)


def _combine_body(p_ref, sp_ref, o_ref):
    o_ref[...] = jnp.maximum(p_ref[0] + p_ref[1] + sp_ref[...], 0.0)


def _full(shape):
    return pl.BlockSpec(shape, lambda i: tuple(0 for _ in shape))


def _proj1(h, b2, m, ws, blk):
    n, d = h.shape
    emb = ws.shape[1]
    r_emb = m.shape[1]
    grid = n // blk
    return pl.pallas_call(
        _proj_body,
        grid=(grid,),
        in_specs=[
            pl.BlockSpec((blk, d), lambda i: (i, 0)),
            _full(b2.shape),
            _full(m.shape),
            _full(ws.shape),
        ],
        out_specs=[
            pl.BlockSpec((blk, r_emb), lambda i: (i, 0)),
            pl.BlockSpec((blk, emb), lambda i: (i, 0)),
        ],
        out_shape=[
            jax.ShapeDtypeStruct((n, r_emb), F32),
            jax.ShapeDtypeStruct((n, emb), F32),
        ],
    )(h, b2, m, ws)


def _proj2(p, sp1, b2, m, ws, blk):
    n = sp1.shape[0]
    emb = ws.shape[1]
    r_emb = m.shape[1]
    grid = n // blk
    return pl.pallas_call(
        _proj2_body,
        grid=(grid,),
        in_specs=[
            pl.BlockSpec((2, blk, emb), lambda i: (0, i, 0)),
            pl.BlockSpec((blk, emb), lambda i: (i, 0)),
            _full(b2.shape),
            _full(m.shape),
            _full(ws.shape),
        ],
        out_specs=[
            pl.BlockSpec((blk, r_emb), lambda i: (i, 0)),
            pl.BlockSpec((blk, emb), lambda i: (i, 0)),
        ],
        out_shape=[
            jax.ShapeDtypeStruct((n, r_emb), F32),
            jax.ShapeDtypeStruct((n, emb), F32),
        ],
    )(p, sp1, b2, m, ws)


def _combine(p, sp, blk):
    n, emb = sp.shape
    grid = n // blk
    return pl.pallas_call(
        _combine_body,
        grid=(grid,),
        in_specs=[
            pl.BlockSpec((2, blk, emb), lambda i: (0, i, 0)),
            pl.BlockSpec((blk, emb), lambda i: (i, 0)),
        ],
        out_specs=pl.BlockSpec((blk, emb), lambda i: (i, 0)),
        out_shape=jax.ShapeDtypeStruct((n, emb), F32),
    )(p, sp)


def _make_sc_agg(n_nodes, nch, emb):
    agg_rows = n_nodes + NS
    zps = agg_rows // NS
    rps = n_nodes // NS
    mesh = plsc.VectorSubcoreMesh(core_axis_name="c", subcore_axis_name="s")

    @functools.partial(
        pl.kernel,
        out_type=jax.ShapeDtypeStruct((NC, n_nodes, emb), F32),
        mesh=mesh,
        scratch_types=[
            pltpu.VMEM((nch, CHUNK), jnp.int32),
            pltpu.VMEM((nch, CHUNK), jnp.int32),
            pltpu.VMEM((nch, CHUNK), jnp.int32),
            pltpu.VMEM((CHUNK, emb), F32),
            pltpu.VMEM_SHARED((agg_rows, emb), F32),
            pltpu.SemaphoreType.DMA,
        ],
    )
    def sc_agg(hr_hbm, srcp_hbm, etyp_hbm, dstp_hbm, zeros_hbm, out_hbm,
               gidx_v, ety_v, dst_v, rows_v, agg_sh, gsem):
        cid = lax.axis_index("c")
        sid = lax.axis_index("s")
        wid = sid * NC + cid

        pltpu.sync_copy(srcp_hbm.at[wid], gidx_v)
        pltpu.sync_copy(etyp_hbm.at[wid], ety_v)
        pltpu.sync_copy(dstp_hbm.at[wid], dst_v)
        pltpu.sync_copy(zeros_hbm.at[pl.ds(sid * zps, zps)],
                        agg_sh.at[pl.ds(sid * zps, zps)])

        def gbody(j, carry):
            for k in range(CHUNK // 16):
                s = gidx_v[j, pl.ds(k * 16, 16)]
                t = ety_v[j, pl.ds(k * 16, 16)]
                gidx_v[j, pl.ds(k * 16, 16)] = s * 8 + t
            return carry

        lax.fori_loop(0, nch, gbody, 0)
        plsc.subcore_barrier()

        def mbody(j, carry):
            pltpu.async_copy(hr_hbm.at[gidx_v.at[j]], rows_v, gsem).wait()
            pltpu.sync_copy(rows_v, agg_sh.at[dst_v.at[j]], add=True)
            return carry

        lax.fori_loop(0, nch, mbody, 0)
        plsc.subcore_barrier()

        pltpu.sync_copy(agg_sh.at[pl.ds(sid * rps, rps)],
                        out_hbm.at[cid, pl.ds(sid * rps, rps)])

    return sc_agg


def kernel(x, edge_index, edge_type, w_in_bases, w_comp1, w_self1,
           w_bases2, w_comp2, w_self2):
    n, inp = x.shape
    nb, _, emb = w_in_bases.shape
    nr = w_comp1.shape[0]
    e = edge_type.shape[0]

    epw = -(-e // NW)
    epw = -(-epw // CHUNK) * CHUNK
    nch = epw // CHUNK
    e_pad = epw * NW
    pad = e_pad - e

    src = edge_index[0]
    dst = edge_index[1]
    zi = jnp.zeros((pad,), jnp.int32)
    srcp = jnp.concatenate([src, zi]).reshape(NW, nch, CHUNK)
    etyp = jnp.concatenate([edge_type, zi]).reshape(NW, nch, CHUNK)
    dstp = jnp.concatenate([dst, jnp.full((pad,), n, jnp.int32)]
                           ).reshape(NW, nch, CHUNK)
    zeros = jnp.zeros((n + NS, emb), F32)

    eye = jnp.eye(emb, dtype=F32)
    b1 = jnp.transpose(w_in_bases, (1, 0, 2)).reshape(inp, nb * emb)
    m1 = (w_comp1.T[:, None, :, None] * eye[None, :, None, :]
          ).reshape(nb * emb, nr * emb)
    b2 = jnp.transpose(w_bases2, (1, 0, 2)).reshape(emb, nb * emb)
    m2 = (w_comp2.T[:, None, :, None] * eye[None, :, None, :]
          ).reshape(nb * emb, nr * emb)

    sc_agg = _make_sc_agg(n, nch, emb)
    blk = 1000

    hs1, sp1 = _proj1(x, b1, m1, w_self1, blk)
    p1 = sc_agg(hs1.reshape(n * nr, emb), srcp, etyp, dstp, zeros)
    hs2, sp2 = _proj2(p1, sp1, b2, m2, w_self2, blk)
    p2 = sc_agg(hs2.reshape(n * nr, emb), srcp, etyp, dstp, zeros)
    return _combine(p2, sp2, blk)


# speedup vs baseline: 13.6312x; 13.6312x over previous
import functools

import jax
import jax.numpy as jnp
from jax import lax
from jax.experimental import pallas as pl
from jax.experimental.pallas import tpu as pltpu
from jax.experimental.pallas import tpu_sc as plsc

NC = 2
NS = 16
NW = NC * NS
CHUNK = 128

F32 = jnp.float32


def _proj_body(h_ref, b_ref, m_ref, ws_ref, hs_ref, sp_ref):
    h = h_ref[...]
    hb = jnp.dot(h, b_ref[...], preferred_element_type=F32)
    hs_ref[...] = jnp.dot(hb, m_ref[...], preferred_element_type=F32)
    sp_ref[...] = jnp.dot(h, ws_ref[...], preferred_element_type=F32)


def _proj2_body(p_ref, sp1_ref, b_ref, m_ref, ws_ref, hs_ref, sp_ref):
    h = jnp.maximum(p_ref[0] + p_ref[1] + sp1_ref[...], 0.0)
    hb = jnp.dot(h, b_ref[...], preferred_element_type=F32)
    hs_ref[...] = jnp.dot(hb, m_ref[...], preferred_element_type=F32)
    sp_ref[...] = jnp.dot(h, ws_ref[...], preferred_element_type=F32)


def _combine_body(p_ref, sp_ref, o_ref):
    o_ref[...] = jnp.maximum(p_ref[0] + p_ref[1] + sp_ref[...], 0.0)


def _full(shape):
    return pl.BlockSpec(shape, lambda i: tuple(0 for _ in shape))


def _proj1(h, b2, m, ws, blk):
    n, d = h.shape
    emb = ws.shape[1]
    r_emb = m.shape[1]
    grid = n // blk
    return pl.pallas_call(
        _proj_body,
        grid=(grid,),
        in_specs=[
            pl.BlockSpec((blk, d), lambda i: (i, 0)),
            _full(b2.shape),
            _full(m.shape),
            _full(ws.shape),
        ],
        out_specs=[
            pl.BlockSpec((blk, r_emb), lambda i: (i, 0)),
            pl.BlockSpec((blk, emb), lambda i: (i, 0)),
        ],
        out_shape=[
            jax.ShapeDtypeStruct((n, r_emb), F32),
            jax.ShapeDtypeStruct((n, emb), F32),
        ],
    )(h, b2, m, ws)


def _proj2(p, sp1, b2, m, ws, blk):
    n = sp1.shape[0]
    emb = ws.shape[1]
    r_emb = m.shape[1]
    grid = n // blk
    return pl.pallas_call(
        _proj2_body,
        grid=(grid,),
        in_specs=[
            pl.BlockSpec((2, blk, emb), lambda i: (0, i, 0)),
            pl.BlockSpec((blk, emb), lambda i: (i, 0)),
            _full(b2.shape),
            _full(m.shape),
            _full(ws.shape),
        ],
        out_specs=[
            pl.BlockSpec((blk, r_emb), lambda i: (i, 0)),
            pl.BlockSpec((blk, emb), lambda i: (i, 0)),
        ],
        out_shape=[
            jax.ShapeDtypeStruct((n, r_emb), F32),
            jax.ShapeDtypeStruct((n, emb), F32),
        ],
    )(p, sp1, b2, m, ws)


def _combine(p, sp, blk):
    n, emb = sp.shape
    grid = n // blk
    return pl.pallas_call(
        _combine_body,
        grid=(grid,),
        in_specs=[
            pl.BlockSpec((2, blk, emb), lambda i: (0, i, 0)),
            pl.BlockSpec((blk, emb), lambda i: (i, 0)),
        ],
        out_specs=pl.BlockSpec((blk, emb), lambda i: (i, 0)),
        out_shape=jax.ShapeDtypeStruct((n, emb), F32),
    )(p, sp)


def _make_sc_agg(n_nodes, nch, emb):
    agg_rows = -(-(n_nodes + 1) // 128) * 128
    zps = agg_rows // NS
    mesh = plsc.VectorSubcoreMesh(core_axis_name="c", subcore_axis_name="s")

    @functools.partial(
        pl.kernel,
        out_type=jax.ShapeDtypeStruct((NC, agg_rows, emb), F32),
        mesh=mesh,
        scratch_types=[
            pltpu.VMEM((nch, CHUNK), jnp.int32),
            pltpu.VMEM((nch, CHUNK), jnp.int32),
            pltpu.VMEM((nch, CHUNK), jnp.int32),
            pltpu.VMEM((CHUNK, emb), F32),
            pltpu.VMEM_SHARED((agg_rows, emb), F32),
            pltpu.SemaphoreType.DMA,
        ],
        compiler_params=pltpu.CompilerParams(use_tc_tiling_on_sc=False),
    )
    def sc_agg(hr_hbm, srcp_hbm, etyp_hbm, dstp_hbm, zeros_hbm, out_hbm,
               gidx_v, ety_v, dst_v, rows_v, agg_sh, gsem):
        cid = lax.axis_index("c")
        sid = lax.axis_index("s")
        wid = sid * NC + cid

        pltpu.sync_copy(srcp_hbm.at[wid], gidx_v)
        pltpu.sync_copy(etyp_hbm.at[wid], ety_v)
        pltpu.sync_copy(dstp_hbm.at[wid], dst_v)
        pltpu.sync_copy(zeros_hbm.at[pl.ds(sid * zps, zps)],
                        agg_sh.at[pl.ds(sid * zps, zps)])

        def gbody(j, carry):
            for k in range(CHUNK // 16):
                s = gidx_v[j, pl.ds(k * 16, 16)]
                t = ety_v[j, pl.ds(k * 16, 16)]
                gidx_v[j, pl.ds(k * 16, 16)] = s * 8 + t
            return carry

        lax.fori_loop(0, nch, gbody, 0)
        plsc.subcore_barrier()

        def mbody(j, carry):
            pltpu.async_copy(hr_hbm.at[gidx_v.at[j]], rows_v, gsem).wait()
            pltpu.sync_copy(rows_v, agg_sh.at[dst_v.at[j]], add=True)
            return carry

        lax.fori_loop(0, nch, mbody, 0)
        plsc.subcore_barrier()

        pltpu.sync_copy(agg_sh.at[pl.ds(sid * zps, zps)],
                        out_hbm.at[cid, pl.ds(sid * zps, zps)])

    return sc_agg


def kernel(x, edge_index, edge_type, w_in_bases, w_comp1, w_self1,
           w_bases2, w_comp2, w_self2):
    n, inp = x.shape
    nb, _, emb = w_in_bases.shape
    nr = w_comp1.shape[0]
    e = edge_type.shape[0]

    epw = -(-e // NW)
    epw = -(-epw // CHUNK) * CHUNK
    nch = epw // CHUNK
    e_pad = epw * NW
    pad = e_pad - e

    src = edge_index[0]
    dst = edge_index[1]
    zi = jnp.zeros((pad,), jnp.int32)
    srcp = jnp.concatenate([src, zi]).reshape(NW, nch, CHUNK)
    etyp = jnp.concatenate([edge_type, zi]).reshape(NW, nch, CHUNK)
    dstp = jnp.concatenate([dst, jnp.full((pad,), n, jnp.int32)]
                           ).reshape(NW, nch, CHUNK)
    zeros = jnp.zeros((-(-(n + 1) // 128) * 128, emb), F32)

    eye = jnp.eye(emb, dtype=F32)
    b1 = jnp.transpose(w_in_bases, (1, 0, 2)).reshape(inp, nb * emb)
    m1 = (w_comp1.T[:, None, :, None] * eye[None, :, None, :]
          ).reshape(nb * emb, nr * emb)
    b2 = jnp.transpose(w_bases2, (1, 0, 2)).reshape(emb, nb * emb)
    m2 = (w_comp2.T[:, None, :, None] * eye[None, :, None, :]
          ).reshape(nb * emb, nr * emb)

    sc_agg = _make_sc_agg(n, nch, emb)
    blk = 1000

    hs1, sp1 = _proj1(x, b1, m1, w_self1, blk)
    p1 = sc_agg(hs1.reshape(n * nr, emb), srcp, etyp, dstp, zeros)
    hs2, sp2 = _proj2(p1, sp1, b2, m2, w_self2, blk)
    p2 = sc_agg(hs2.reshape(n * nr, emb), srcp, etyp, dstp, zeros)
    return _combine(p2, sp2, blk)


# speedup vs baseline: 13.6519x; 1.0015x over previous
import functools

import jax
import jax.numpy as jnp
from jax import lax
from jax.experimental import pallas as pl
from jax.experimental.pallas import tpu as pltpu
from jax.experimental.pallas import tpu_sc as plsc

NC = 2
NS = 16
NW = NC * NS
CHUNK = 128
NBUF = 1

F32 = jnp.float32


def _proj_body(h_ref, b_ref, m_ref, ws_ref, hs_ref, sp_ref):
    h = h_ref[...]
    hb = jnp.dot(h, b_ref[...], preferred_element_type=F32)
    hs_ref[...] = jnp.dot(hb, m_ref[...], preferred_element_type=F32)
    sp_ref[...] = jnp.dot(h, ws_ref[...], preferred_element_type=F32)


def _proj2_body(p_ref, sp1_ref, b_ref, m_ref, ws_ref, hs_ref, sp_ref):
    h = jnp.maximum(p_ref[0] + p_ref[1] + sp1_ref[...], 0.0)
    hb = jnp.dot(h, b_ref[...], preferred_element_type=F32)
    hs_ref[...] = jnp.dot(hb, m_ref[...], preferred_element_type=F32)
    sp_ref[...] = jnp.dot(h, ws_ref[...], preferred_element_type=F32)


def _combine_body(p_ref, sp_ref, o_ref):
    o_ref[...] = jnp.maximum(p_ref[0] + p_ref[1] + sp_ref[...], 0.0)


def _full(shape):
    return pl.BlockSpec(shape, lambda i: tuple(0 for _ in shape))


def _proj1(h, b2, m, ws, blk):
    n, d = h.shape
    emb = ws.shape[1]
    r_emb = m.shape[1]
    grid = n // blk
    return pl.pallas_call(
        _proj_body,
        grid=(grid,),
        in_specs=[
            pl.BlockSpec((blk, d), lambda i: (i, 0)),
            _full(b2.shape),
            _full(m.shape),
            _full(ws.shape),
        ],
        out_specs=[
            pl.BlockSpec((blk, r_emb), lambda i: (i, 0)),
            pl.BlockSpec((blk, emb), lambda i: (i, 0)),
        ],
        out_shape=[
            jax.ShapeDtypeStruct((n, r_emb), F32),
            jax.ShapeDtypeStruct((n, emb), F32),
        ],
    )(h, b2, m, ws)


def _proj2(p, sp1, b2, m, ws, blk):
    n = sp1.shape[0]
    emb = ws.shape[1]
    r_emb = m.shape[1]
    grid = n // blk
    return pl.pallas_call(
        _proj2_body,
        grid=(grid,),
        in_specs=[
            pl.BlockSpec((2, blk, emb), lambda i: (0, i, 0)),
            pl.BlockSpec((blk, emb), lambda i: (i, 0)),
            _full(b2.shape),
            _full(m.shape),
            _full(ws.shape),
        ],
        out_specs=[
            pl.BlockSpec((blk, r_emb), lambda i: (i, 0)),
            pl.BlockSpec((blk, emb), lambda i: (i, 0)),
        ],
        out_shape=[
            jax.ShapeDtypeStruct((n, r_emb), F32),
            jax.ShapeDtypeStruct((n, emb), F32),
        ],
    )(p, sp1, b2, m, ws)


def _combine(p, sp, blk):
    n, emb = sp.shape
    grid = n // blk
    return pl.pallas_call(
        _combine_body,
        grid=(grid,),
        in_specs=[
            pl.BlockSpec((2, blk, emb), lambda i: (0, i, 0)),
            pl.BlockSpec((blk, emb), lambda i: (i, 0)),
        ],
        out_specs=pl.BlockSpec((blk, emb), lambda i: (i, 0)),
        out_shape=jax.ShapeDtypeStruct((n, emb), F32),
    )(p, sp)


def _make_sc_agg(n_nodes, nch, emb):
    agg_rows = -(-(n_nodes + 1) // 128) * 128
    zps = agg_rows // NS
    mesh = plsc.VectorSubcoreMesh(core_axis_name="c", subcore_axis_name="s")

    @functools.partial(
        pl.kernel,
        out_type=jax.ShapeDtypeStruct((NC, agg_rows, emb), F32),
        mesh=mesh,
        scratch_types=[
            pltpu.VMEM((nch, CHUNK), jnp.int32),
            pltpu.VMEM((nch, CHUNK), jnp.int32),
            pltpu.VMEM((nch, CHUNK), jnp.int32),
            pltpu.VMEM((CHUNK, emb), F32),
            pltpu.VMEM_SHARED((agg_rows, emb), F32),
            pltpu.SemaphoreType.DMA,
        ],
        compiler_params=pltpu.CompilerParams(use_tc_tiling_on_sc=False),
    )
    def sc_agg(hr_hbm, srcp_hbm, etyp_hbm, dstp_hbm, zeros_hbm, out_hbm,
               gidx_v, ety_v, dst_v, rows_v, agg_sh, gsem):
        cid = lax.axis_index("c")
        sid = lax.axis_index("s")
        wid = sid * NC + cid

        pltpu.sync_copy(srcp_hbm.at[wid], gidx_v)
        pltpu.sync_copy(etyp_hbm.at[wid], ety_v)
        pltpu.sync_copy(dstp_hbm.at[wid], dst_v)
        pltpu.sync_copy(zeros_hbm.at[pl.ds(sid * zps, zps)],
                        agg_sh.at[pl.ds(sid * zps, zps)])

        def gbody(j, carry):
            for k in range(CHUNK // 16):
                s = gidx_v[j, pl.ds(k * 16, 16)]
                t = ety_v[j, pl.ds(k * 16, 16)]
                gidx_v[j, pl.ds(k * 16, 16)] = s * 8 + t
            return carry

        lax.fori_loop(0, nch, gbody, 0)
        plsc.subcore_barrier()

        def mbody(j, carry):
            pltpu.async_copy(hr_hbm.at[gidx_v.at[j]], rows_v, gsem).wait()
            pltpu.sync_copy(rows_v, agg_sh.at[dst_v.at[j]], add=True)
            return carry

        lax.fori_loop(0, nch, mbody, 0)
        plsc.subcore_barrier()

        pltpu.sync_copy(agg_sh.at[pl.ds(sid * zps, zps)],
                        out_hbm.at[cid, pl.ds(sid * zps, zps)])

    return sc_agg


def kernel(x, edge_index, edge_type, w_in_bases, w_comp1, w_self1,
           w_bases2, w_comp2, w_self2):
    n, inp = x.shape
    nb, _, emb = w_in_bases.shape
    nr = w_comp1.shape[0]
    e = edge_type.shape[0]

    epw = -(-e // NW)
    epw = -(-epw // (CHUNK * NBUF)) * (CHUNK * NBUF)
    nch = epw // CHUNK
    e_pad = epw * NW
    pad = e_pad - e

    src = edge_index[0]
    dst = edge_index[1]
    zi = jnp.zeros((pad,), jnp.int32)
    srcp = jnp.concatenate([src, zi]).reshape(NW, nch, CHUNK)
    etyp = jnp.concatenate([edge_type, zi]).reshape(NW, nch, CHUNK)
    dstp = jnp.concatenate([dst, jnp.full((pad,), n, jnp.int32)]
                           ).reshape(NW, nch, CHUNK)
    zeros = jnp.zeros((-(-(n + 1) // 128) * 128, emb), F32)

    eye = jnp.eye(emb, dtype=F32)
    b1 = jnp.transpose(w_in_bases, (1, 0, 2)).reshape(inp, nb * emb)
    m1 = (w_comp1.T[:, None, :, None] * eye[None, :, None, :]
          ).reshape(nb * emb, nr * emb)
    b2 = jnp.transpose(w_bases2, (1, 0, 2)).reshape(emb, nb * emb)
    m2 = (w_comp2.T[:, None, :, None] * eye[None, :, None, :]
          ).reshape(nb * emb, nr * emb)

    sc_agg = _make_sc_agg(n, nch, emb)
    blk = 1000

    hs1, sp1 = _proj1(x, b1, m1, w_self1, blk)
    p1 = sc_agg(hs1.reshape(n * nr, emb), srcp, etyp, dstp, zeros)
    hs2, sp2 = _proj2(p1, sp1, b2, m2, w_self2, blk)
    p2 = sc_agg(hs2.reshape(n * nr, emb), srcp, etyp, dstp, zeros)
    return _combine(p2, sp2, blk)
